# baseline (speedup 1.0000x reference)
import jax
import jax.numpy as jnp
from jax import lax
from jax.experimental import pallas as pl
from jax.experimental.pallas import tpu as pltpu
from jax.experimental.pallas import tpu_sc as plsc

_NC = 2
_NS = 16
_L = 16
_NW = _NC * _NS
_B = 16384
_K = 32
_BPW = _B // _NW
_NG = _BPW // _L
_NCOLS = 7813
_WC = 4
_NWIN = 62
_CAP = 832
_NVC = _B // _L


def _phase_a(u_hbm, i_hbm, gut_hbm, git_hbm, gu_flat, gi_flat,
             allidx_v, cand_u, cand_j, win_ring, sval, sidx, sem, wsem):
    wid = lax.axis_index("s") * _NC + lax.axis_index("c")
    ncols = jnp.where(wid < 5, 245, 244)
    c0 = wid * 244 + jnp.minimum(wid, 5)
    rlo = c0 * 128
    rhi = rlo + ncols * 128
    lanes = lax.iota(jnp.int32, _L)

    for idx_hbm, tab_hbm, out_flat in ((u_hbm, gut_hbm, gu_flat),
                                       (i_hbm, git_hbm, gi_flat)):
        pltpu.sync_copy(idx_hbm, allidx_v)

        neg1 = jnp.full((_L,), -1, jnp.int32)

        def prefill(v, _):
            sidx[pl.ds(v * _L, _L)] = neg1
            return 0

        lax.fori_loop(0, _CAP * _K // _L, prefill, 0)

        def collect(v, tot):
            vec = allidx_v[pl.ds(v * _L, _L)]
            mask = (vec >= rlo) & (vec < rhi)
            plsc.store_compressed(cand_u.at[pl.ds(tot, _L)], vec, mask=mask)
            plsc.store_compressed(cand_j.at[pl.ds(tot, _L)],
                                  lanes + v * _L, mask=mask)
            return tot + plsc.all_reduce_population_count(mask)[0]

        ncand = lax.fori_loop(0, _NVC, collect, jnp.int32(0))
        nvc_c = (ncand + _L - 1) // _L

        def fire(t, slot):
            fc = jnp.minimum(c0 + _WC * t, c0 + ncols - _WC)
            off = pl.multiple_of(fc * 128, 128)
            pltpu.make_async_copy(
                tab_hbm.at[:, pl.ds(off, _WC * 128)],
                win_ring.at[slot], wsem).start()

        def wait_win(slot):
            pltpu.make_async_copy(
                tab_hbm.at[:, pl.ds(0, _WC * 128)],
                win_ring.at[slot], wsem).wait()

        fire(0, 0)
        fire(1, 1)

        def window(t, carry):
            n = carry
            slot = t & 1
            wait_win(slot)
            fc = jnp.minimum(c0 + _WC * t, c0 + ncols - _WC)
            lo = rlo + t * (_WC * 128)
            hi = jnp.minimum(lo + _WC * 128, rhi)
            wbase = fc * 128

            def wcollect(v, st):
                m, base = st
                vec = cand_u[pl.ds(v * _L, _L)]
                jj = cand_j[pl.ds(v * _L, _L)]
                mask = ((vec >= lo) & (vec < hi)
                        & ((lanes + v * _L) < ncand))
                plsc.store_compressed(cand_u.at[pl.ds(_CAP + m, _L)],
                                      vec, mask=mask)
                plsc.store_compressed(cand_j.at[pl.ds(_CAP + m, _L)],
                                      jj, mask=mask)
                return (m + plsc.all_reduce_population_count(mask)[0], base)

            wcnt, _ = lax.fori_loop(0, nvc_c, wcollect, (jnp.int32(0), 0))

            def extract(e, nn):
                sel = pl.ds((_CAP + ((e >> 4) << 4)), _L)
                pick = lanes == (e & 15)
                uu = jnp.sum(jnp.where(pick, cand_u[sel], 0))
                jj = jnp.sum(jnp.where(pick, cand_j[sel], 0))
                lane = jnp.full((_L,), uu - wbase, jnp.int32)
                va = plsc.load_gather(win_ring.at[slot], [lanes, lane])
                vb = plsc.load_gather(win_ring.at[slot], [lanes + _L, lane])
                sval[pl.ds(nn * _K, _L)] = va
                sval[pl.ds(nn * _K + _L, _L)] = vb
                sidx[pl.ds(nn * _K, _L)] = jj * _K + lanes
                sidx[pl.ds(nn * _K + _L, _L)] = jj * _K + _L + lanes
                return nn + 1

            n = lax.fori_loop(0, wcnt, extract, n)

            @pl.when(t < _NWIN - 2)
            def _():
                fire(t + 2, slot)

            return n

        lax.fori_loop(0, _NWIN, window, jnp.int32(0))

        pltpu.async_copy(
            sval, out_flat.at[plsc.Indices(sidx, ignored_value=-1)], sem
        ).wait()


def _phase_b(u_hbm, i_hbm, r_hbm, a_hbm, al_hbm, ag_hbm,
             bu_hbm, bi_hbm, gu_flat, gi_flat, out_hbm,
             idxu_v, idxi_v, r_v, a_v, al_v, ag_v,
             bu_v, bi_v, gu_v, gi_v, tbuf, accbuf, sem):
    wid = lax.axis_index("s") * _NC + lax.axis_index("c")
    base = wid * _BPW

    pltpu.sync_copy(u_hbm.at[pl.ds(base, _BPW)], idxu_v)
    pltpu.sync_copy(i_hbm.at[pl.ds(base, _BPW)], idxi_v)
    pltpu.sync_copy(r_hbm.at[pl.ds(base, _BPW)], r_v)
    pltpu.sync_copy(a_hbm.at[pl.ds(base, _BPW)], a_v)
    pltpu.sync_copy(al_hbm, al_v)
    pltpu.sync_copy(ag_hbm, ag_v)
    pltpu.sync_copy(gu_flat.at[pl.ds(base * _K, _BPW * _K)], gu_v)
    pltpu.sync_copy(gi_flat.at[pl.ds(base * _K, _BPW * _K)], gi_v)

    bcopies = [
        pltpu.async_copy(bu_hbm.at[idxu_v], bu_v, sem),
        pltpu.async_copy(bi_hbm.at[idxi_v], bi_v, sem),
    ]
    for c in bcopies:
        c.wait()

    alpha = al_v[...]
    agec = ag_v[...]
    lanes = lax.iota(jnp.int32, _L)
    scat_base = lanes * _L

    def group(g, acc):
        row0 = g * _L
        for s in range(_L):
            ro = (row0 + s) * _K
            w = (gu_v[pl.ds(ro, _L)] * gi_v[pl.ds(ro, _L)]
                 + gu_v[pl.ds(ro + _L, _L)] * gi_v[pl.ds(ro + _L, _L)])
            plsc.store_scatter(tbuf, [scat_base + s], w)
        dot = tbuf[pl.ds(0, _L)]
        for d in range(1, _L):
            dot = dot + tbuf[pl.ds(d * _L, _L)]
        chunk = pl.ds(row0, _L)
        diff = (alpha + bu_v[chunk] + bi_v[chunk] + dot
                + a_v[chunk] * agec - r_v[chunk])
        return acc + diff * diff

    acc = lax.fori_loop(0, _NG, group, jnp.zeros((_L,), jnp.float32))
    accbuf[...] = acc
    pltpu.sync_copy(accbuf, out_hbm.at[wid])


@jax.jit
def _lfm_sc(sampleU, sampleI, sampleR, sampleA, al16, ag16,
            betaU, betaI, gammaU_t, gammaI_t):
    mesh = plsc.VectorSubcoreMesh(core_axis_name="c", subcore_axis_name="s")
    params = pltpu.CompilerParams(
        needs_layout_passes=False, use_tc_tiling_on_sc=True)
    gu_flat, gi_flat = pl.kernel(
        _phase_a,
        out_type=(jax.ShapeDtypeStruct((_B * _K,), jnp.float32),
                  jax.ShapeDtypeStruct((_B * _K,), jnp.float32)),
        mesh=mesh,
        compiler_params=params,
        scratch_types=[
            pltpu.VMEM((_B,), jnp.int32),
            pltpu.VMEM((2 * _CAP + _L,), jnp.int32),
            pltpu.VMEM((2 * _CAP + _L,), jnp.int32),
            pltpu.VMEM((2, _K, _WC * 128), jnp.float32),
            pltpu.VMEM((_CAP * _K,), jnp.float32),
            pltpu.VMEM((_CAP * _K,), jnp.int32),
            pltpu.SemaphoreType.DMA,
            pltpu.SemaphoreType.DMA,
        ],
    )(sampleU, sampleI, gammaU_t, gammaI_t)

    return pl.kernel(
        _phase_b,
        out_type=jax.ShapeDtypeStruct((_NW, _L), jnp.float32),
        mesh=mesh,
        compiler_params=params,
        scratch_types=[
            pltpu.VMEM((_BPW,), jnp.int32),
            pltpu.VMEM((_BPW,), jnp.int32),
            pltpu.VMEM((_BPW,), jnp.float32),
            pltpu.VMEM((_BPW,), jnp.float32),
            pltpu.VMEM((_L,), jnp.float32),
            pltpu.VMEM((_L,), jnp.float32),
            pltpu.VMEM((_BPW,), jnp.float32),
            pltpu.VMEM((_BPW,), jnp.float32),
            pltpu.VMEM((_BPW * _K,), jnp.float32),
            pltpu.VMEM((_BPW * _K,), jnp.float32),
            pltpu.VMEM((_L * _L,), jnp.float32),
            pltpu.VMEM((_L,), jnp.float32),
            pltpu.SemaphoreType.DMA,
        ],
    )(sampleU, sampleI, sampleR, sampleA, al16, ag16,
      betaU, betaI, gu_flat, gi_flat)


def kernel(sampleU, sampleI, sampleR, sampleA, alpha, agecoef,
           betaU, betaI, gammaU, gammaI):
    al16 = jnp.full((_L,), alpha, jnp.float32)
    ag16 = jnp.full((_L,), agecoef, jnp.float32)
    partials = _lfm_sc(sampleU, sampleI, sampleR, sampleA, al16, ag16,
                       betaU, betaI, gammaU.T, gammaI.T)
    return jnp.sum(partials) * (1.0 / _B)
